# baseline (speedup 1.0000x reference)
import jax
import jax.numpy as jnp
from jax import lax
from jax.experimental import pallas as pl
from jax.experimental.pallas import tpu as pltpu
from jax.experimental.pallas import tpu_sc as plsc

GROUP = 8
EPS = 1e-05
NSUB = 16
S1C = 384


def _pass_a(x_ref, w1_ref, b1_ref, b2_ref, w2_ref, lt_ref, mom_ref, w12t_ref):
    i = pl.program_id(0)

    @pl.when(i == 0)
    def _():
        w12t_ref[...] = jax.lax.dot_general(
            w2_ref[...], w1_ref[...],
            (((0,), (1,)), ((), ())),
            preferred_element_type=jnp.float32)

    xb = x_ref[0]
    b12 = jnp.sum(w12t_ref[...] * b1_ref[...], axis=1,
                  keepdims=True) + b2_ref[...]
    lt_ref[...] = jnp.dot(w12t_ref[...], xb,
                          preferred_element_type=jnp.float32) + b12
    csum = jnp.sum(xb, axis=0, keepdims=True)
    cssq = jnp.sum(xb * xb, axis=0, keepdims=True)
    mom_ref[...] = jnp.concatenate([csum, cssq], axis=0)


def _argmax16(lt_v, base):
    sl = pl.ds(base, 16)
    m = lt_v[0, sl]
    idx = jnp.zeros((16,), jnp.int32)
    for g in range(1, GROUP):
        v = lt_v[g, sl]
        gt = v > m
        m = jnp.where(gt, v, m)
        idx = jnp.where(gt, jnp.full((16,), g, jnp.int32), idx)
    return idx


def _rsqrt16(a):
    y = plsc.bitcast(
        jnp.full((16,), 0x5F3759DF, jnp.int32)
        - lax.shift_right_logical(plsc.bitcast(a, jnp.int32),
                                  jnp.full((16,), 1, jnp.int32)),
        jnp.float32)
    for _ in range(4):
        y = y * (1.5 - 0.5 * a * y * y)
    return y


def _sc_route(lt_hbm, mom_hbm, w_hbm, b_hbm, scale_hbm, off_hbm,
              lt_v, mom_v, idx_v, bins_v, gacc_v, red_v, wb_v, out_v, shared):
    sid = lax.axis_index("s")
    c0 = sid * S1C
    n1 = sid // 2
    c1 = (sid % 2) * S1C

    pltpu.sync_copy(lt_hbm.at[:, pl.ds(c0, S1C)], lt_v)
    pltpu.sync_copy(mom_hbm.at[:, pl.ds(c0, S1C)], mom_v)
    z = jnp.zeros((16,), jnp.float32)
    bins_v[pl.ds(0, 16)] = z
    bins_v[pl.ds(16, 16)] = z
    ones = jnp.ones((16,), jnp.float32)
    for v in range(S1C // 16):
        sl = pl.ds(v * 16, 16)
        idx = _argmax16(lt_v, v * 16)
        idx_v[sl] = idx
        plsc.addupdate_scatter(bins_v, [idx], ones)
        plsc.addupdate_scatter(bins_v, [idx + 8], mom_v[0, sl])
        plsc.addupdate_scatter(bins_v, [idx + 16], mom_v[1, sl])

    pltpu.sync_copy(bins_v, shared.at[sid])
    plsc.subcore_barrier()
    pltpu.sync_copy(shared, gacc_v)
    acc0 = jnp.zeros((16,), jnp.float32)
    acc1 = jnp.zeros((16,), jnp.float32)
    for t in range(NSUB):
        acc0 = acc0 + gacc_v[t, pl.ds(0, 16)]
        acc1 = acc1 + gacc_v[t, pl.ds(16, 16)]
    red_v[pl.ds(0, 16)] = acc0
    red_v[pl.ds(16, 16)] = acc1
    lane = lax.iota(jnp.int32, 16)
    idx8 = lax.rem(lane, jnp.full((16,), 8, jnp.int32))
    cnt = plsc.load_gather(red_v, [idx8])
    ssum = plsc.load_gather(red_v, [idx8 + 8])
    sssq = plsc.load_gather(red_v, [idx8 + 16])
    total = cnt * 1024.0
    mean = ssum / jnp.maximum(total, 1.0)
    sq = sssq - ssum * mean
    var = sq / jnp.maximum(total - 1.0, 1.0)
    rstd = _rsqrt16(var + EPS)
    red_v[pl.ds(0, 16)] = mean
    red_v[pl.ds(16, 16)] = rstd

    pltpu.sync_copy(w_hbm.at[pl.ds(c1, S1C)], wb_v.at[pl.ds(0, S1C)])
    pltpu.sync_copy(b_hbm.at[pl.ds(c1, S1C)], wb_v.at[pl.ds(S1C, S1C)])
    for v in range(S1C // 16):
        sl = pl.ds(v * 16, 16)
        idx = idx_v[sl]
        mean_c = plsc.load_gather(red_v, [idx])
        rstd_c = plsc.load_gather(red_v, [idx + 16])
        sc = rstd_c * wb_v[sl]
        out_v[sl] = sc
        out_v[pl.ds(S1C + v * 16, 16)] = (
            wb_v[pl.ds(S1C + v * 16, 16)] - mean_c * sc)
    pltpu.sync_copy(out_v.at[pl.ds(0, S1C)],
                    scale_hbm.at[n1, pl.ds(c1, S1C)])
    pltpu.sync_copy(out_v.at[pl.ds(S1C, S1C)],
                    off_hbm.at[n1, pl.ds(c1, S1C)])


def _pass_c(x_ref, sc_ref, of_ref, out_ref):
    i = pl.program_id(0)
    sc = sc_ref[pl.ds(i, 1), :]
    of = of_ref[pl.ds(i, 1), :]
    out_ref[...] = x_ref[0] * sc + of


def kernel(x, W1, b1, W2, b2, weight, bias):
    n, c, h, w = x.shape
    hw = h * w
    xt = jnp.transpose(x, (0, 2, 3, 1)).reshape(n, hw, c)

    lt, mom = pl.pallas_call(
        _pass_a,
        grid=(n,),
        in_specs=[
            pl.BlockSpec((1, hw, c), lambda i: (i, 0, 0)),
            pl.BlockSpec((hw, hw), lambda i: (0, 0)),
            pl.BlockSpec((1, hw), lambda i: (0, 0)),
            pl.BlockSpec((GROUP, 1), lambda i: (0, 0)),
            pl.BlockSpec((hw, GROUP), lambda i: (0, 0)),
        ],
        out_specs=[
            pl.BlockSpec((GROUP, c), lambda i: (0, i)),
            pl.BlockSpec((2, c), lambda i: (0, i)),
        ],
        out_shape=[
            jax.ShapeDtypeStruct((GROUP, n * c), jnp.float32),
            jax.ShapeDtypeStruct((2, n * c), jnp.float32),
        ],
        scratch_shapes=[pltpu.VMEM((GROUP, hw), jnp.float32)],
    )(xt, W1, b1.reshape(1, hw), b2.reshape(GROUP, 1), W2)

    mesh = plsc.VectorSubcoreMesh(core_axis_name="c", subcore_axis_name="s")
    scale, off = pl.kernel(
        _sc_route,
        mesh=mesh,
        compiler_params=pltpu.CompilerParams(needs_layout_passes=False),
        out_type=[
            jax.ShapeDtypeStruct((n, c), jnp.float32),
            jax.ShapeDtypeStruct((n, c), jnp.float32),
        ],
        scratch_types=[
            pltpu.VMEM((GROUP, S1C), jnp.float32),
            pltpu.VMEM((2, S1C), jnp.float32),
            pltpu.VMEM((S1C,), jnp.int32),
            pltpu.VMEM((32,), jnp.float32),
            pltpu.VMEM((NSUB, 32), jnp.float32),
            pltpu.VMEM((32,), jnp.float32),
            pltpu.VMEM((2 * S1C,), jnp.float32),
            pltpu.VMEM((2 * S1C,), jnp.float32),
            pltpu.VMEM_SHARED((NSUB, 32), jnp.float32),
        ],
    )(lt, mom, weight.reshape(c), bias.reshape(c))

    out2 = pl.pallas_call(
        _pass_c,
        grid=(n,),
        in_specs=[
            pl.BlockSpec((1, hw, c), lambda i: (i, 0, 0)),
            pl.BlockSpec((n, c), lambda i: (0, 0)),
            pl.BlockSpec((n, c), lambda i: (0, 0)),
        ],
        out_specs=pl.BlockSpec((hw, c), lambda i: (i, 0)),
        out_shape=jax.ShapeDtypeStruct((n * hw, c), jnp.float32),
    )(xt, scale, off)

    return jnp.transpose(out2.reshape(n, h, w, c), (0, 3, 1, 2))


# speedup vs baseline: 2.2564x; 2.2564x over previous
import jax
import jax.numpy as jnp
from jax.experimental import pallas as pl
from jax.experimental.pallas import tpu as pltpu

GROUP = 8
EPS = 1e-05


def _fused(x_ref, w1_ref, b1_ref, b2_ref, w2_ref, w_ref, b_ref, out_ref,
           w12t_ref, cache_ref, oh_ref, stats_ref, scale_ref, off_ref):
    p = pl.program_id(0)
    i = pl.program_id(1)
    n = pl.num_programs(1)
    hw = x_ref.shape[1]

    @pl.when((p == 0) & (i == 0))
    def _():
        w12t_ref[0:GROUP] = jax.lax.dot_general(
            w2_ref[...], w1_ref[...],
            (((0,), (1,)), ((), ())),
            preferred_element_type=jnp.float32)
        w12t_ref[GROUP:GROUP + 1] = jnp.ones_like(w12t_ref[GROUP:GROUP + 1])
        stats_ref[...] = jnp.zeros_like(stats_ref)

    @pl.when(p == 0)
    def _():
        xb = x_ref[0]
        cache_ref[i] = xb
        b12 = jnp.sum(w12t_ref[0:GROUP] * b1_ref[...], axis=1,
                      keepdims=True) + b2_ref[...]
        y = jnp.dot(w12t_ref[...], xb,
                    preferred_element_type=jnp.float32)
        lt = y[0:GROUP] + b12
        csum = y[GROUP:GROUP + 1]
        mx = jnp.max(lt, axis=0, keepdims=True)
        rowid = jax.lax.broadcasted_iota(jnp.int32, lt.shape, 0)
        idx = jnp.min(jnp.where(lt >= mx, rowid, GROUP), axis=0, keepdims=True)
        oh = (rowid == idx).astype(jnp.float32)
        oh_ref[i] = oh

        cssq = jnp.dot(w12t_ref[GROUP:GROUP + 1], xb * xb,
                       preferred_element_type=jnp.float32)
        cnt_g = jnp.sum(oh, axis=1, keepdims=True)
        sum_g = jnp.sum(oh * csum, axis=1, keepdims=True)
        ssq_g = jnp.sum(oh * cssq, axis=1, keepdims=True)
        stats_ref[...] += jnp.concatenate([cnt_g, sum_g, ssq_g], axis=1)

    @pl.when((p == 1) & (i == 0))
    def _():
        cnt_rows = stats_ref[:, 0:1]
        total = cnt_rows * float(hw)
        s = stats_ref[:, 1:2]
        q = stats_ref[:, 2:3]
        mean = s / jnp.maximum(total, 1.0)
        sq = q - s * mean
        var = sq / jnp.maximum(total - 1.0, 1.0)
        rstd = jax.lax.rsqrt(var + EPS)
        for k in range(n):
            oh = oh_ref[k]
            rstd_c = jnp.sum(oh * rstd, axis=0, keepdims=True)
            mean_c = jnp.sum(oh * mean, axis=0, keepdims=True)
            sc = rstd_c * w_ref[...]
            scale_ref[k] = sc
            off_ref[k] = b_ref[...] - mean_c * sc

    @pl.when(p == 1)
    def _():
        out_ref[...] = cache_ref[i] * scale_ref[i] + off_ref[i]


def kernel(x, W1, b1, W2, b2, weight, bias):
    n, c, h, w = x.shape
    hw = h * w
    xt = jnp.transpose(x, (0, 2, 3, 1)).reshape(n, hw, c)

    out2 = pl.pallas_call(
        _fused,
        grid=(2, n),
        in_specs=[
            pl.BlockSpec((1, hw, c), lambda p, i: (jnp.where(p == 0, i, n - 1), 0, 0)),
            pl.BlockSpec((hw, hw), lambda p, i: (0, 0)),
            pl.BlockSpec((1, hw), lambda p, i: (0, 0)),
            pl.BlockSpec((GROUP, 1), lambda p, i: (0, 0)),
            pl.BlockSpec((hw, GROUP), lambda p, i: (0, 0)),
            pl.BlockSpec((1, c), lambda p, i: (0, 0)),
            pl.BlockSpec((1, c), lambda p, i: (0, 0)),
        ],
        out_specs=pl.BlockSpec((hw, c), lambda p, i: (jnp.where(p == 0, 0, i), 0)),
        out_shape=jax.ShapeDtypeStruct((n * hw, c), jnp.float32),
        scratch_shapes=[
            pltpu.VMEM((GROUP + 1, hw), jnp.float32),
            pltpu.VMEM((n, hw, c), jnp.float32),
            pltpu.VMEM((n, GROUP, c), jnp.float32),
            pltpu.VMEM((GROUP, 3), jnp.float32),
            pltpu.VMEM((n, 1, c), jnp.float32),
            pltpu.VMEM((n, 1, c), jnp.float32),
        ],
    )(xt, W1, b1.reshape(1, hw), b2.reshape(GROUP, 1), W2,
      weight.reshape(1, c), bias.reshape(1, c))

    return jnp.transpose(out2.reshape(n, h, w, c), (0, 3, 1, 2))


# speedup vs baseline: 2.2935x; 1.0164x over previous
import jax
import jax.numpy as jnp
from jax.experimental import pallas as pl
from jax.experimental.pallas import tpu as pltpu

GROUP = 8
EPS = 1e-05


def _fused(x_ref, w1_ref, b1_ref, b2_ref, w2_ref, w_ref, b_ref, out_ref,
           w12t_ref, cache_ref, oh_ref, stats_ref, scale_ref, off_ref):
    p = pl.program_id(0)
    i = pl.program_id(1)
    n = pl.num_programs(1)
    hw = x_ref.shape[1]

    @pl.when((p == 0) & (i == 0))
    def _():
        w12t_ref[...] = jax.lax.dot_general(
            w2_ref[...], w1_ref[...],
            (((0,), (1,)), ((), ())),
            preferred_element_type=jnp.float32)
        stats_ref[...] = jnp.zeros_like(stats_ref)

    @pl.when(p == 0)
    def _():
        xb = x_ref[0]
        cache_ref[i] = xb
        b12 = jnp.sum(w12t_ref[...] * b1_ref[...], axis=1,
                      keepdims=True) + b2_ref[...]
        lt = jnp.dot(w12t_ref[...], xb,
                     preferred_element_type=jnp.float32) + b12
        mx = jnp.max(lt, axis=0, keepdims=True)
        rowid = jax.lax.broadcasted_iota(jnp.int32, lt.shape, 0)
        idx = jnp.min(jnp.where(lt >= mx, rowid, GROUP), axis=0, keepdims=True)
        oh = (rowid == idx).astype(jnp.float32)
        oh_ref[i] = oh

        csum = jnp.sum(xb, axis=0, keepdims=True)
        cssq = jnp.sum(xb * xb, axis=0, keepdims=True)
        cnt_g = jnp.sum(oh, axis=1, keepdims=True)
        sum_g = jnp.sum(oh * csum, axis=1, keepdims=True)
        ssq_g = jnp.sum(oh * cssq, axis=1, keepdims=True)
        stats_ref[...] += jnp.concatenate([cnt_g, sum_g, ssq_g], axis=1)

    @pl.when((p == 1) & (i == 0))
    def _():
        cnt_rows = stats_ref[:, 0:1]
        total = cnt_rows * float(hw)
        s = stats_ref[:, 1:2]
        q = stats_ref[:, 2:3]
        mean = s / jnp.maximum(total, 1.0)
        sq = q - s * mean
        var = sq / jnp.maximum(total - 1.0, 1.0)
        rstd = jax.lax.rsqrt(var + EPS)
        for k in range(n):
            oh = oh_ref[k]
            rstd_c = jnp.sum(oh * rstd, axis=0, keepdims=True)
            mean_c = jnp.sum(oh * mean, axis=0, keepdims=True)
            sc = rstd_c * w_ref[...]
            scale_ref[k] = sc
            off_ref[k] = b_ref[...] - mean_c * sc

    @pl.when(p == 1)
    def _():
        out_ref[...] = cache_ref[i] * scale_ref[i] + off_ref[i]


def kernel(x, W1, b1, W2, b2, weight, bias):
    n, c, h, w = x.shape
    hw = h * w
    xt = jnp.transpose(x, (0, 2, 3, 1)).reshape(n, hw, c)

    out2 = pl.pallas_call(
        _fused,
        grid=(2, n),
        in_specs=[
            pl.BlockSpec((1, hw, c), lambda p, i: (jnp.where(p == 0, i, n - 1), 0, 0)),
            pl.BlockSpec((hw, hw), lambda p, i: (0, 0)),
            pl.BlockSpec((1, hw), lambda p, i: (0, 0)),
            pl.BlockSpec((GROUP, 1), lambda p, i: (0, 0)),
            pl.BlockSpec((hw, GROUP), lambda p, i: (0, 0)),
            pl.BlockSpec((1, c), lambda p, i: (0, 0)),
            pl.BlockSpec((1, c), lambda p, i: (0, 0)),
        ],
        out_specs=pl.BlockSpec((hw, c), lambda p, i: (jnp.where(p == 0, 0, i), 0)),
        out_shape=jax.ShapeDtypeStruct((n * hw, c), jnp.float32),
        scratch_shapes=[
            pltpu.VMEM((GROUP, hw), jnp.float32),
            pltpu.VMEM((n, hw, c), jnp.float32),
            pltpu.VMEM((n, GROUP, c), jnp.float32),
            pltpu.VMEM((GROUP, 3), jnp.float32),
            pltpu.VMEM((n, 1, c), jnp.float32),
            pltpu.VMEM((n, 1, c), jnp.float32),
        ],
    )(xt, W1, b1.reshape(1, hw), b2.reshape(GROUP, 1), W2,
      weight.reshape(1, c), bias.reshape(1, c))

    return jnp.transpose(out2.reshape(n, h, w, c), (0, 3, 1, 2))
